# baseline (device time: 28650 ns/iter reference)
import jax
import jax.numpy as jnp
from jax import lax
from jax.experimental import pallas as pl
from jax.experimental.pallas import tpu as pltpu

B, HS, WS, C = 2, 64, 64, 64
GN = 128 * 128


def kernel(x, k, Wp):
    def body(x_ref, k_ref, wp_ref, out_ref,
             p_ref, stats_ref, rs_send, rs_recv, cs_send, cs_recv,
             send_sems, recv_sems):
        my_x = lax.axis_index("x")
        my_y = lax.axis_index("y")
        x_nbr = (1 - my_x, my_y)
        y_nbr = (my_x, 1 - my_y)

        barrier = pltpu.get_barrier_semaphore()
        pl.semaphore_signal(barrier, inc=1, device_id=x_nbr,
                            device_id_type=pl.DeviceIdType.MESH)
        pl.semaphore_signal(barrier, inc=1, device_id=y_nbr,
                            device_id_type=pl.DeviceIdType.MESH)
        pl.semaphore_wait(barrier, 2)

        xv = x_ref[...]
        xb = xv.astype(jnp.bfloat16)

        s = jnp.sum(xv, axis=(1, 2))
        ss = jnp.sum(xv * xv, axis=(1, 2))
        stats_ref[0, :, :] = jnp.concatenate([s, ss], axis=0)

        stats_x = pltpu.make_async_remote_copy(
            src_ref=stats_ref.at[0], dst_ref=stats_ref.at[1],
            send_sem=send_sems.at[0], recv_sem=recv_sems.at[0],
            device_id=x_nbr, device_id_type=pl.DeviceIdType.MESH)
        stats_x.start()

        @pl.when(my_x == 0)
        def _():
            rs_send[...] = xb[:, HS - 1:HS, :, :]

        @pl.when(my_x == 1)
        def _():
            rs_send[...] = xb[:, 0:1, :, :]

        row = pltpu.make_async_remote_copy(
            src_ref=rs_send, dst_ref=rs_recv,
            send_sem=send_sems.at[2], recv_sem=recv_sems.at[2],
            device_id=x_nbr, device_id_type=pl.DeviceIdType.MESH)
        row.start()

        p_ref[:, 1:HS + 1, 1:WS + 1, :] = xb

        @pl.when(my_x == 0)
        def _():
            p_ref[:, 0:1, 1:WS + 1, :] = xb[:, 0:1, :, :]

        @pl.when(my_x == 1)
        def _():
            p_ref[:, HS + 1:HS + 2, 1:WS + 1, :] = xb[:, HS - 1:HS, :, :]

        stats_x.wait()
        stats_ref[2, :, :] = stats_ref[0, :, :] + stats_ref[1, :, :]
        stats_y = pltpu.make_async_remote_copy(
            src_ref=stats_ref.at[2], dst_ref=stats_ref.at[3],
            send_sem=send_sems.at[1], recv_sem=recv_sems.at[1],
            device_id=y_nbr, device_id_type=pl.DeviceIdType.MESH)
        stats_y.start()

        row.wait()

        @pl.when(my_x == 0)
        def _():
            p_ref[:, HS + 1:HS + 2, 1:WS + 1, :] = rs_recv[...]

        @pl.when(my_x == 1)
        def _():
            p_ref[:, 0:1, 1:WS + 1, :] = rs_recv[...]

        @pl.when(my_y == 0)
        def _():
            cs_send[...] = p_ref[:, :, WS:WS + 1, :]
            p_ref[:, :, 0:1, :] = p_ref[:, :, 1:2, :]

        @pl.when(my_y == 1)
        def _():
            cs_send[...] = p_ref[:, :, 1:2, :]
            p_ref[:, :, WS + 1:WS + 2, :] = p_ref[:, :, WS:WS + 1, :]

        col = pltpu.make_async_remote_copy(
            src_ref=cs_send, dst_ref=cs_recv,
            send_sem=send_sems.at[3], recv_sem=recv_sems.at[3],
            device_id=y_nbr, device_id_type=pl.DeviceIdType.MESH)
        col.start()

        col.wait()

        @pl.when(my_y == 0)
        def _():
            p_ref[:, :, WS + 1:WS + 2, :] = cs_recv[...]

        @pl.when(my_y == 1)
        def _():
            p_ref[:, :, 0:1, :] = cs_recv[...]

        pv = p_ref[...]
        kv = k_ref[...]
        kb = kv.astype(jnp.bfloat16)
        conv_raw = jnp.zeros((B, HS, WS, C), jnp.float32)
        for di in range(3):
            for dj in range(3):
                conv_raw = conv_raw + (
                    pv[:, di:di + HS, dj:dj + WS, :]
                    * kb[di, dj][None, None, None, :]).astype(jnp.float32)

        stats_y.wait()
        tot = stats_ref[2, :, :] + stats_ref[3, :, :]
        mean = tot[0:B] * (1.0 / GN)
        var = tot[B:2 * B] * (1.0 / GN) - mean * mean
        inv = lax.rsqrt(var + 1e-5)
        ksum = jnp.sum(kv, axis=(0, 1))
        beta = -mean * inv * ksum[None, :]

        conv = (conv_raw * inv[:, None, None, :]
                + beta[:, None, None, :])
        a = conv * (1.0 / (1.0 + jnp.exp(-conv)))
        proj = jnp.dot(a.astype(jnp.bfloat16).reshape(B * HS * WS, C),
                       wp_ref[...].astype(jnp.bfloat16),
                       preferred_element_type=jnp.float32)
        out_ref[...] = xv + proj.reshape(B, HS, WS, C)

    return pl.pallas_call(
        body,
        out_shape=jax.ShapeDtypeStruct((B, HS, WS, C), jnp.float32),
        in_specs=[
            pl.BlockSpec(memory_space=pltpu.VMEM),
            pl.BlockSpec(memory_space=pltpu.VMEM),
            pl.BlockSpec(memory_space=pltpu.VMEM),
        ],
        out_specs=pl.BlockSpec(memory_space=pltpu.VMEM),
        scratch_shapes=[
            pltpu.VMEM((B, HS + 2, WS + 2, C), jnp.bfloat16),
            pltpu.VMEM((4, 2 * B, C), jnp.float32),
            pltpu.VMEM((B, 1, WS, C), jnp.bfloat16),
            pltpu.VMEM((B, 1, WS, C), jnp.bfloat16),
            pltpu.VMEM((B, HS + 2, 1, C), jnp.bfloat16),
            pltpu.VMEM((B, HS + 2, 1, C), jnp.bfloat16),
            pltpu.SemaphoreType.DMA((4,)),
            pltpu.SemaphoreType.DMA((4,)),
        ],
        compiler_params=pltpu.CompilerParams(collective_id=0),
    )(x, k, Wp)


# device time: 23676 ns/iter; 1.2101x vs baseline; 1.2101x over previous
import jax
import jax.numpy as jnp
from jax import lax
from jax.experimental import pallas as pl
from jax.experimental.pallas import tpu as pltpu

B, HS, WS, C = 2, 64, 64, 64
GN = 128 * 128


def kernel(x, k, Wp):
    def body(x_ref, k_ref, wp_ref, out_ref,
             p_ref, stats_ref, rs_send, rs_recv, cs_send, cs_recv,
             send_sems, recv_sems):
        my_x = lax.axis_index("x")
        my_y = lax.axis_index("y")
        x_nbr = (1 - my_x, my_y)
        y_nbr = (my_x, 1 - my_y)

        barrier = pltpu.get_barrier_semaphore()
        pl.semaphore_signal(barrier, inc=1, device_id=x_nbr,
                            device_id_type=pl.DeviceIdType.MESH)
        pl.semaphore_signal(barrier, inc=1, device_id=y_nbr,
                            device_id_type=pl.DeviceIdType.MESH)
        pl.semaphore_wait(barrier, 2)

        xv = x_ref[...]
        xb = xv

        s = jnp.sum(xv, axis=(1, 2))
        ss = jnp.sum(xv * xv, axis=(1, 2))
        stats_ref[0, :, :] = jnp.concatenate([s, ss], axis=0)

        stats_x = pltpu.make_async_remote_copy(
            src_ref=stats_ref.at[0], dst_ref=stats_ref.at[1],
            send_sem=send_sems.at[0], recv_sem=recv_sems.at[0],
            device_id=x_nbr, device_id_type=pl.DeviceIdType.MESH)
        stats_x.start()

        @pl.when(my_x == 0)
        def _():
            rs_send[...] = xb[:, HS - 1:HS, :, :]

        @pl.when(my_x == 1)
        def _():
            rs_send[...] = xb[:, 0:1, :, :]

        row = pltpu.make_async_remote_copy(
            src_ref=rs_send, dst_ref=rs_recv,
            send_sem=send_sems.at[2], recv_sem=recv_sems.at[2],
            device_id=x_nbr, device_id_type=pl.DeviceIdType.MESH)
        row.start()

        p_ref[:, 1:HS + 1, 1:WS + 1, :] = xb

        @pl.when(my_x == 0)
        def _():
            p_ref[:, 0:1, 1:WS + 1, :] = xb[:, 0:1, :, :]

        @pl.when(my_x == 1)
        def _():
            p_ref[:, HS + 1:HS + 2, 1:WS + 1, :] = xb[:, HS - 1:HS, :, :]

        stats_x.wait()
        stats_ref[2, :, :] = stats_ref[0, :, :] + stats_ref[1, :, :]
        stats_y = pltpu.make_async_remote_copy(
            src_ref=stats_ref.at[2], dst_ref=stats_ref.at[3],
            send_sem=send_sems.at[1], recv_sem=recv_sems.at[1],
            device_id=y_nbr, device_id_type=pl.DeviceIdType.MESH)
        stats_y.start()

        row.wait()

        @pl.when(my_x == 0)
        def _():
            p_ref[:, HS + 1:HS + 2, 1:WS + 1, :] = rs_recv[...]

        @pl.when(my_x == 1)
        def _():
            p_ref[:, 0:1, 1:WS + 1, :] = rs_recv[...]

        @pl.when(my_y == 0)
        def _():
            cs_send[...] = p_ref[:, :, WS:WS + 1, :]
            p_ref[:, :, 0:1, :] = p_ref[:, :, 1:2, :]

        @pl.when(my_y == 1)
        def _():
            cs_send[...] = p_ref[:, :, 1:2, :]
            p_ref[:, :, WS + 1:WS + 2, :] = p_ref[:, :, WS:WS + 1, :]

        col = pltpu.make_async_remote_copy(
            src_ref=cs_send, dst_ref=cs_recv,
            send_sem=send_sems.at[3], recv_sem=recv_sems.at[3],
            device_id=y_nbr, device_id_type=pl.DeviceIdType.MESH)
        col.start()

        col.wait()

        @pl.when(my_y == 0)
        def _():
            p_ref[:, :, WS + 1:WS + 2, :] = cs_recv[...]

        @pl.when(my_y == 1)
        def _():
            p_ref[:, :, 0:1, :] = cs_recv[...]

        pv = p_ref[...]
        kv = k_ref[...]
        conv_raw = jnp.zeros((B, HS, WS, C), jnp.float32)
        for di in range(3):
            for dj in range(3):
                conv_raw = conv_raw + (
                    pv[:, di:di + HS, dj:dj + WS, :]
                    * kv[di, dj][None, None, None, :])

        stats_y.wait()
        tot = stats_ref[2, :, :] + stats_ref[3, :, :]
        mean = tot[0:B] * (1.0 / GN)
        var = tot[B:2 * B] * (1.0 / GN) - mean * mean
        inv = lax.rsqrt(var + 1e-5)
        ksum = jnp.sum(kv, axis=(0, 1))
        beta = -mean * inv * ksum[None, :]

        conv = (conv_raw * inv[:, None, None, :]
                + beta[:, None, None, :])
        a = conv * (1.0 / (1.0 + jnp.exp(-conv)))
        proj = jnp.dot(a.astype(jnp.bfloat16).reshape(B * HS * WS, C),
                       wp_ref[...].astype(jnp.bfloat16),
                       preferred_element_type=jnp.float32)
        out_ref[...] = xv + proj.reshape(B, HS, WS, C)

    return pl.pallas_call(
        body,
        out_shape=jax.ShapeDtypeStruct((B, HS, WS, C), jnp.float32),
        in_specs=[
            pl.BlockSpec(memory_space=pltpu.VMEM),
            pl.BlockSpec(memory_space=pltpu.VMEM),
            pl.BlockSpec(memory_space=pltpu.VMEM),
        ],
        out_specs=pl.BlockSpec(memory_space=pltpu.VMEM),
        scratch_shapes=[
            pltpu.VMEM((B, HS + 2, WS + 2, C), jnp.float32),
            pltpu.VMEM((4, 2 * B, C), jnp.float32),
            pltpu.VMEM((B, 1, WS, C), jnp.float32),
            pltpu.VMEM((B, 1, WS, C), jnp.float32),
            pltpu.VMEM((B, HS + 2, 1, C), jnp.float32),
            pltpu.VMEM((B, HS + 2, 1, C), jnp.float32),
            pltpu.SemaphoreType.DMA((4,)),
            pltpu.SemaphoreType.DMA((4,)),
        ],
        compiler_params=pltpu.CompilerParams(collective_id=0),
    )(x, k, Wp)


# device time: 17271 ns/iter; 1.6589x vs baseline; 1.3709x over previous
import jax
import jax.numpy as jnp
from jax import lax
from jax.experimental import pallas as pl
from jax.experimental.pallas import tpu as pltpu

B, HS, WS, C = 2, 64, 64, 64
GN = 128 * 128


def kernel(x, k, Wp):
    def body(x_ref, k_ref, wp_ref, out_ref,
             p_ref, stats_ref, rs_send, rs_recv, cs_send, cs_recv,
             dg_send, dg_recv, yc_send, yc_recv, send_sems, recv_sems):
        my_x = lax.axis_index("x")
        my_y = lax.axis_index("y")
        x_nbr = (1 - my_x, my_y)
        y_nbr = (my_x, 1 - my_y)
        dg_nbr = (1 - my_x, 1 - my_y)

        barrier = pltpu.get_barrier_semaphore()
        for nbr in (x_nbr, y_nbr, dg_nbr):
            pl.semaphore_signal(barrier, inc=1, device_id=nbr,
                                device_id_type=pl.DeviceIdType.MESH)
        pl.semaphore_wait(barrier, 3)

        src_row = jnp.where(my_x == 0, HS - 1, 0)
        dst_row = jnp.where(my_x == 0, HS + 1, 0)
        ep_row = jnp.where(my_x == 0, 0, HS + 1)
        out_row = jnp.where(my_x == 0, 0, HS - 1)
        in_row = jnp.where(my_x == 0, HS - 1, 0)
        rr_d = jnp.where(my_x == 0, HS + 1, 0)
        rr_y = jnp.where(my_x == 0, 0, HS + 1)

        def ds(i):
            return pl.ds(i, 1)

        ALL = slice(None)

        rs_send[...] = x_ref[ALL, ds(src_row), ALL, ALL]
        row = pltpu.make_async_remote_copy(
            src_ref=rs_send, dst_ref=rs_recv,
            send_sem=send_sems.at[0], recv_sem=recv_sems.at[0],
            device_id=x_nbr, device_id_type=pl.DeviceIdType.MESH)
        row.start()

        @pl.when(my_y == 0)
        def _():
            cs_send[...] = x_ref[:, :, WS - 1:WS, :]
            dg_send[...] = x_ref[ALL, ds(in_row), ds(WS - 1), ALL]
            yc_send[...] = x_ref[ALL, ds(out_row), ds(WS - 1), ALL]

        @pl.when(my_y == 1)
        def _():
            cs_send[...] = x_ref[:, :, 0:1, :]
            dg_send[...] = x_ref[ALL, ds(in_row), ds(0), ALL]
            yc_send[...] = x_ref[ALL, ds(out_row), ds(0), ALL]

        colint = pltpu.make_async_remote_copy(
            src_ref=cs_send, dst_ref=cs_recv,
            send_sem=send_sems.at[1], recv_sem=recv_sems.at[1],
            device_id=y_nbr, device_id_type=pl.DeviceIdType.MESH)
        colint.start()
        diag = pltpu.make_async_remote_copy(
            src_ref=dg_send, dst_ref=dg_recv,
            send_sem=send_sems.at[2], recv_sem=recv_sems.at[2],
            device_id=dg_nbr, device_id_type=pl.DeviceIdType.MESH)
        diag.start()
        yc = pltpu.make_async_remote_copy(
            src_ref=yc_send, dst_ref=yc_recv,
            send_sem=send_sems.at[3], recv_sem=recv_sems.at[3],
            device_id=y_nbr, device_id_type=pl.DeviceIdType.MESH)
        yc.start()

        xv = x_ref[...]
        x2d = xv.reshape(B * HS * WS, C)
        ones_blk = jnp.concatenate(
            [jnp.concatenate([jnp.ones((1, HS * WS), jnp.float32),
                              jnp.zeros((1, HS * WS), jnp.float32)], axis=1),
             jnp.concatenate([jnp.zeros((1, HS * WS), jnp.float32),
                              jnp.ones((1, HS * WS), jnp.float32)], axis=1)],
            axis=0)
        s = jnp.dot(ones_blk, x2d, preferred_element_type=jnp.float32)
        ss = jnp.dot(ones_blk, x2d * x2d,
                     preferred_element_type=jnp.float32)
        stats_ref[0, :, :] = jnp.concatenate([s, ss], axis=0)
        stats_rdmas = []
        for i, nbr in enumerate((x_nbr, y_nbr, dg_nbr)):
            r = pltpu.make_async_remote_copy(
                src_ref=stats_ref.at[0], dst_ref=stats_ref.at[i + 1],
                send_sem=send_sems.at[4 + i], recv_sem=recv_sems.at[4 + i],
                device_id=nbr, device_id_type=pl.DeviceIdType.MESH)
            r.start()
            stats_rdmas.append(r)

        p_ref[:, 1:HS + 1, 1:WS + 1, :] = xv
        p_ref[ALL, ds(ep_row), pl.ds(1, WS), ALL] = (
            x_ref[ALL, ds(out_row), ALL, ALL])

        row.wait()
        p_ref[ALL, ds(dst_row), pl.ds(1, WS), ALL] = rs_recv[...]

        colint.wait()
        diag.wait()
        yc.wait()

        @pl.when(my_y == 0)
        def _():
            p_ref[:, :, 0:1, :] = p_ref[:, :, 1:2, :]
            p_ref[ALL, pl.ds(1, HS), ds(WS + 1), ALL] = cs_recv[...]
            p_ref[ALL, ds(rr_d), ds(WS + 1), ALL] = dg_recv[...]
            p_ref[ALL, ds(rr_y), ds(WS + 1), ALL] = yc_recv[...]

        @pl.when(my_y == 1)
        def _():
            p_ref[:, :, WS + 1:WS + 2, :] = p_ref[:, :, WS:WS + 1, :]
            p_ref[ALL, pl.ds(1, HS), ds(0), ALL] = cs_recv[...]
            p_ref[ALL, ds(rr_d), ds(0), ALL] = dg_recv[...]
            p_ref[ALL, ds(rr_y), ds(0), ALL] = yc_recv[...]

        for r in stats_rdmas:
            r.wait()
        tot = (stats_ref[0, :, :] + stats_ref[1, :, :]
               + stats_ref[2, :, :] + stats_ref[3, :, :])
        mean = tot[0:B] * (1.0 / GN)
        var = tot[B:2 * B] * (1.0 / GN) - mean * mean
        inv = lax.rsqrt(var + 1e-5)

        pv = p_ref[...]
        kv = k_ref[...]
        ksum = jnp.sum(kv, axis=(0, 1))
        beta = mean * inv * ksum[None, :]
        conv = jnp.broadcast_to(-beta[:, None, None, :], (B, HS, WS, C))
        for di in range(3):
            for dj in range(3):
                w_t = kv[di, dj][None, :] * inv
                conv = conv + (pv[:, di:di + HS, dj:dj + WS, :]
                               * w_t[:, None, None, :])
        a = conv * jax.nn.sigmoid(conv)
        proj = jnp.dot(a.reshape(B * HS * WS, C), wp_ref[...],
                       preferred_element_type=jnp.float32)
        out_ref[...] = xv + proj.reshape(B, HS, WS, C)

    return pl.pallas_call(
        body,
        out_shape=jax.ShapeDtypeStruct((B, HS, WS, C), jnp.float32),
        in_specs=[
            pl.BlockSpec(memory_space=pltpu.VMEM),
            pl.BlockSpec(memory_space=pltpu.VMEM),
            pl.BlockSpec(memory_space=pltpu.VMEM),
        ],
        out_specs=pl.BlockSpec(memory_space=pltpu.VMEM),
        scratch_shapes=[
            pltpu.VMEM((B, HS + 2, WS + 2, C), jnp.float32),
            pltpu.VMEM((4, 2 * B, C), jnp.float32),
            pltpu.VMEM((B, 1, WS, C), jnp.float32),
            pltpu.VMEM((B, 1, WS, C), jnp.float32),
            pltpu.VMEM((B, HS, 1, C), jnp.float32),
            pltpu.VMEM((B, HS, 1, C), jnp.float32),
            pltpu.VMEM((B, 1, 1, C), jnp.float32),
            pltpu.VMEM((B, 1, 1, C), jnp.float32),
            pltpu.VMEM((B, 1, 1, C), jnp.float32),
            pltpu.VMEM((B, 1, 1, C), jnp.float32),
            pltpu.SemaphoreType.DMA((7,)),
            pltpu.SemaphoreType.DMA((7,)),
        ],
        compiler_params=pltpu.CompilerParams(collective_id=0),
    )(x, k, Wp)
